# baseline (speedup 1.0000x reference)
import functools

import jax
import jax.numpy as jnp
from jax import lax
from jax.experimental import pallas as pl
from jax.experimental.pallas import tpu as pltpu
from jax.experimental.pallas import tpu_sc as plsc

NUM_LANGUAGES = 100000
EMBED_DIM = 64
BATCH = 16384

NC = 2
NS = 16
NW = NC * NS
B_PER_W = BATCH // NW
CHUNK = 128
NCHUNK = B_PER_W // CHUNK
L = 16
GROUPS = B_PER_W // L


def _rsqrt_newton(x):
    i = plsc.bitcast(x, jnp.int32)
    i = jnp.int32(0x5F3759DF) - (i >> 1)
    y = plsc.bitcast(i, jnp.float32)
    half_x = x * 0.5
    for _ in range(3):
        y = y * (1.5 - half_x * y * y)
    return y


def _make_sc_kernel():
    mesh = plsc.VectorSubcoreMesh(core_axis_name="c", subcore_axis_name="s")

    @functools.partial(
        pl.kernel,
        mesh=mesh,
        out_type=jax.ShapeDtypeStruct((BATCH,), jnp.float32),
        compiler_params=pltpu.CompilerParams(
            needs_layout_passes=False, use_tc_tiling_on_sc=False),
        scratch_types=[
            pltpu.VMEM((NCHUNK, CHUNK), jnp.int32),
            pltpu.VMEM((NCHUNK, CHUNK), jnp.int32),
            pltpu.VMEM((B_PER_W, EMBED_DIM), jnp.float32),
            pltpu.VMEM((B_PER_W, EMBED_DIM), jnp.float32),
            pltpu.VMEM((B_PER_W,), jnp.float32),
            pltpu.SemaphoreType.DMA,
        ],
    )
    def sc_kernel(src_idx_hbm, tgt_idx_hbm, table_hbm, out_hbm,
                  idx_s, idx_t, rows_s, rows_t, out_v, sem):
        wid = lax.axis_index("s") * NC + lax.axis_index("c")
        base = wid * B_PER_W

        pltpu.sync_copy(src_idx_hbm.at[wid], idx_s)
        pltpu.sync_copy(tgt_idx_hbm.at[wid], idx_t)

        copies = []
        for j in range(NCHUNK):
            copies.append(pltpu.async_copy(
                table_hbm.at[idx_s.at[j]],
                rows_s.at[pl.ds(j * CHUNK, CHUNK)], sem))
            copies.append(pltpu.async_copy(
                table_hbm.at[idx_t.at[j]],
                rows_t.at[pl.ds(j * CHUNK, CHUNK)], sem))
        for c in copies:
            c.wait()

        eps = jnp.float32(1e-8)
        lanes = jnp.arange(L, dtype=jnp.int32)
        zeros = jnp.zeros((L,), jnp.float32)

        def grp_body(g, _):
            base_row = g * L
            acc_d = zeros
            acc_a = zeros
            acc_b = zeros
            for j in range(L):
                i = base_row + j
                sv0 = rows_s[i, pl.ds(0, L)]
                tv0 = rows_t[i, pl.ds(0, L)]
                dot_p = sv0 * tv0
                n1_p = sv0 * sv0
                n2_p = tv0 * tv0
                for c in range(1, EMBED_DIM // L):
                    sv = rows_s[i, pl.ds(c * L, L)]
                    tv = rows_t[i, pl.ds(c * L, L)]
                    dot_p = dot_p + sv * tv
                    n1_p = n1_p + sv * sv
                    n2_p = n2_p + tv * tv
                mask = lanes == j
                acc_d = jnp.where(mask, jnp.sum(dot_p), acc_d)
                acc_a = jnp.where(mask, jnp.sum(n1_p), acc_a)
                acc_b = jnp.where(mask, jnp.sum(n2_p), acc_b)
            na = acc_a * _rsqrt_newton(acc_a)
            nb = acc_b * _rsqrt_newton(acc_b)
            denom = jnp.maximum(na, eps) * jnp.maximum(nb, eps)
            sim = acc_d / denom
            out_v[pl.ds(base_row, L)] = sim * 0.5 + 0.5
            return _

        lax.fori_loop(0, GROUPS, grp_body, None)

        pltpu.sync_copy(out_v, out_hbm.at[pl.ds(base, B_PER_W)])

    return sc_kernel


_SC_KERNEL = _make_sc_kernel()


@jax.jit
def kernel(source_lang_id, target_lang_id, table):
    src = source_lang_id.astype(jnp.int32).reshape(NW, NCHUNK, CHUNK)
    tgt = target_lang_id.astype(jnp.int32).reshape(NW, NCHUNK, CHUNK)
    return _SC_KERNEL(src, tgt, table)


# speedup vs baseline: 1.3796x; 1.3796x over previous
import functools

import jax
import jax.numpy as jnp
from jax import lax
from jax.experimental import pallas as pl
from jax.experimental.pallas import tpu as pltpu
from jax.experimental.pallas import tpu_sc as plsc

NUM_LANGUAGES = 100000
EMBED_DIM = 64
BATCH = 16384

NC = 2
NS = 16
NW = NC * NS
B_PER_W = BATCH // NW
CH = 128
NCHUNK = B_PER_W // CH
L = 16
CGROUPS = CH // L


def _rsqrt_newton(x):
    i = plsc.bitcast(x, jnp.int32)
    i = jnp.int32(0x5F3759DF) - (i >> 1)
    y = plsc.bitcast(i, jnp.float32)
    half_x = x * 0.5
    for _ in range(3):
        y = y * (1.5 - half_x * y * y)
    return y


def _make_sc_kernel():
    mesh = plsc.VectorSubcoreMesh(core_axis_name="c", subcore_axis_name="s")

    @functools.partial(
        pl.kernel,
        mesh=mesh,
        out_type=jax.ShapeDtypeStruct((BATCH,), jnp.float32),
        compiler_params=pltpu.CompilerParams(needs_layout_passes=False),
        scratch_types=[
            pltpu.VMEM((NCHUNK, CH), jnp.int32),
            pltpu.VMEM((NCHUNK, CH), jnp.int32),
            pltpu.VMEM((CH, 2 * EMBED_DIM), jnp.float32),
            pltpu.VMEM((CH, 2 * EMBED_DIM), jnp.float32),
            pltpu.VMEM((B_PER_W,), jnp.float32),
            pltpu.SemaphoreType.DMA,
        ],
    )
    def sc_kernel(src_idx_hbm, tgt_idx_hbm, table_hbm, drain_hbm, out_hbm,
                  idx_s, idx_t, rows_s, rows_t, out_v, sem):
        wid = lax.axis_index("s") * NC + lax.axis_index("c")
        base = wid * B_PER_W

        pltpu.sync_copy(src_idx_hbm.at[pl.ds(wid * NCHUNK, NCHUNK)], idx_s)
        pltpu.sync_copy(tgt_idx_hbm.at[pl.ds(wid * NCHUNK, NCHUNK)], idx_t)

        eps = jnp.float32(1e-8)
        lanes = jnp.arange(L, dtype=jnp.int32)
        zeros = jnp.zeros((L,), jnp.float32)

        def chunk_body(ci, _):
            def fetch_body(g, _):
                col = g * L
                vs = idx_s[ci, pl.ds(col, L)]
                vt = idx_t[ci, pl.ds(col, L)]
                for j in range(L):
                    k = g * L + j
                    pltpu.async_copy(table_hbm.at[vs[j]],
                                     rows_s.at[k, pl.ds(0, EMBED_DIM)],
                                     sem)
                    pltpu.async_copy(table_hbm.at[vt[j]],
                                     rows_t.at[k, pl.ds(0, EMBED_DIM)],
                                     sem)
                return _

            lax.fori_loop(0, CGROUPS, fetch_body, None)

            pltpu.make_async_copy(drain_hbm, rows_s.at[pl.ds(0, CH // 2)],
                                  sem).wait()
            pltpu.make_async_copy(drain_hbm, rows_t.at[pl.ds(0, CH // 2)],
                                  sem).wait()

            def grp_body(g, _):
                acc_d = zeros
                acc_a = zeros
                acc_b = zeros
                for j in range(L):
                    i = g * L + j
                    sv0 = rows_s[i, pl.ds(0, L)]
                    tv0 = rows_t[i, pl.ds(0, L)]
                    dot_p = sv0 * tv0
                    n1_p = sv0 * sv0
                    n2_p = tv0 * tv0
                    for c in range(1, EMBED_DIM // L):
                        sv = rows_s[i, pl.ds(c * L, L)]
                        tv = rows_t[i, pl.ds(c * L, L)]
                        dot_p = dot_p + sv * tv
                        n1_p = n1_p + sv * sv
                        n2_p = n2_p + tv * tv
                    mask = lanes == j
                    acc_d = jnp.where(mask, jnp.sum(dot_p), acc_d)
                    acc_a = jnp.where(mask, jnp.sum(n1_p), acc_a)
                    acc_b = jnp.where(mask, jnp.sum(n2_p), acc_b)
                na = acc_a * _rsqrt_newton(acc_a)
                nb = acc_b * _rsqrt_newton(acc_b)
                denom = jnp.maximum(na, eps) * jnp.maximum(nb, eps)
                sim = acc_d / denom
                out_v[pl.ds(ci * CH + g * L, L)] = sim * 0.5 + 0.5
                return _

            lax.fori_loop(0, CGROUPS, grp_body, None)
            return _

        lax.fori_loop(0, NCHUNK, chunk_body, None)

        pltpu.sync_copy(out_v, out_hbm.at[pl.ds(base, B_PER_W)])

    return sc_kernel


_SC_KERNEL = _make_sc_kernel()


@jax.jit
def kernel(source_lang_id, target_lang_id, table):
    src = source_lang_id.astype(jnp.int32).reshape(128, 128)
    tgt = target_lang_id.astype(jnp.int32).reshape(128, 128)
    drain = jnp.zeros((CH // 2, 2 * EMBED_DIM), jnp.float32)
    return _SC_KERNEL(src, tgt, table, drain)
